# baseline (speedup 1.0000x reference)
import functools

import jax
import jax.numpy as jnp
from jax import lax
from jax.experimental import pallas as pl
from jax.experimental.pallas import tpu as pltpu
from jax.experimental.pallas import tpu_sc as plsc

_N_SUB = 1_000_000
_N_IDS = 16384
_NS = 16
_CA = 40000
_CB = 22496
_CR = _N_SUB - _NS * _CA - (_NS - 1) * _CB
_SCRATCH = 40192

_mesh = plsc.VectorSubcoreMesh(core_axis_name="c", subcore_axis_name="s")


@functools.partial(
    pl.kernel,
    out_type=jax.ShapeDtypeStruct((_N_SUB,), jnp.float32),
    mesh=_mesh,
    scratch_types=[
        pltpu.VMEM((_N_IDS,), jnp.int32),
        pltpu.VMEM((_SCRATCH,), jnp.float32),
        pltpu.SemaphoreType.DMA,
    ],
    compiler_params=pltpu.CompilerParams(
        needs_layout_passes=False,
        disable_bounds_checks=True,
        disable_semaphore_checks=True,
    ),
)
def _mask_kernel(ids_hbm, out_hbm, ids_v, chunk_v, sem):
    c = lax.axis_index("c")
    s = lax.axis_index("s")
    lo = jnp.where(c == 0, s * _CA, _NS * _CA + s * _CB)
    size = jnp.where(c == 0, _CA, jnp.where(s == _NS - 1, _CR, _CB))
    hi = lo + size

    ids_copy = pltpu.async_copy(ids_hbm, ids_v, sem)

    zero16 = jnp.zeros((16,), jnp.float32)

    @plsc.parallel_loop(0, _SCRATCH, step=256, unroll=2)
    def _(base):
        for j in range(16):
            chunk_v[pl.ds(base + j * 16, 16)] = zero16

    ids_copy.wait()

    ones16 = jnp.full((16,), 1.0, jnp.float32)
    size_u = lax.convert_element_type(size, jnp.uint32)

    @plsc.parallel_loop(0, _N_IDS, step=128, unroll=4)
    def _(base):
        for j in range(8):
            ids16 = ids_v[pl.ds(base + j * 16, 16)]
            local = ids16 - lo
            inb = plsc.bitcast(local, jnp.uint32) < size_u
            plsc.store_scatter(chunk_v, [local], ones16, mask=inb)

    @pl.when(c == 0)
    def _():
        pltpu.sync_copy(chunk_v.at[pl.ds(0, _CA)], out_hbm.at[pl.ds(lo, _CA)])

    @pl.when((c == 1) & (s < _NS - 1))
    def _():
        pltpu.sync_copy(chunk_v.at[pl.ds(0, _CB)], out_hbm.at[pl.ds(lo, _CB)])

    @pl.when((c == 1) & (s == _NS - 1))
    def _():
        pltpu.sync_copy(chunk_v.at[pl.ds(0, _CR)], out_hbm.at[pl.ds(lo, _CR)])


def kernel(subject_ids):
    ids = jnp.reshape(subject_ids, (-1,)).astype(jnp.int32)
    return _mask_kernel(ids)


# speedup vs baseline: 1.0186x; 1.0186x over previous
import functools

import jax
import jax.numpy as jnp
from jax import lax
from jax.experimental import pallas as pl
from jax.experimental.pallas import tpu as pltpu
from jax.experimental.pallas import tpu_sc as plsc

_N_SUB = 1_000_000
_N_IDS = 16384
_NS = 16
_CA = 40000
_CB = 22496
_CR = _N_SUB - _NS * _CA - (_NS - 1) * _CB
_SCRATCH = 40192

_mesh = plsc.VectorSubcoreMesh(core_axis_name="c", subcore_axis_name="s")


@functools.partial(
    pl.kernel,
    out_type=jax.ShapeDtypeStruct((_N_SUB,), jnp.float32),
    mesh=_mesh,
    scratch_types=[
        pltpu.VMEM((_N_IDS,), jnp.int32),
        pltpu.VMEM((_SCRATCH,), jnp.float32),
        pltpu.SemaphoreType.DMA,
    ],
    compiler_params=pltpu.CompilerParams(
        needs_layout_passes=False,
        disable_bounds_checks=True,
        disable_semaphore_checks=True,
    ),
)
def _mask_kernel(ids_hbm, out_hbm, ids_v, chunk_v, sem):
    c = lax.axis_index("c")
    s = lax.axis_index("s")
    lo = jnp.where(c == 1, s * _CA, _NS * _CA + s * _CB)
    size = jnp.where(c == 1, _CA, jnp.where(s == _NS - 1, _CR, _CB))
    hi = lo + size

    ids_copy = pltpu.async_copy(ids_hbm, ids_v, sem)

    zero16 = jnp.zeros((16,), jnp.float32)

    @plsc.parallel_loop(0, _SCRATCH, step=256, unroll=2)
    def _(base):
        for j in range(16):
            chunk_v[pl.ds(base + j * 16, 16)] = zero16

    ids_copy.wait()

    ones16 = jnp.full((16,), 1.0, jnp.float32)
    size_u = lax.convert_element_type(size, jnp.uint32)

    @plsc.parallel_loop(0, _N_IDS, step=128, unroll=4)
    def _(base):
        for j in range(8):
            ids16 = ids_v[pl.ds(base + j * 16, 16)]
            local = ids16 - lo
            inb = plsc.bitcast(local, jnp.uint32) < size_u
            plsc.store_scatter(chunk_v, [local], ones16, mask=inb)

    @pl.when(c == 1)
    def _():
        pltpu.sync_copy(chunk_v.at[pl.ds(0, _CA)], out_hbm.at[pl.ds(lo, _CA)])

    @pl.when((c == 0) & (s < _NS - 1))
    def _():
        pltpu.sync_copy(chunk_v.at[pl.ds(0, _CB)], out_hbm.at[pl.ds(lo, _CB)])

    @pl.when((c == 0) & (s == _NS - 1))
    def _():
        pltpu.sync_copy(chunk_v.at[pl.ds(0, _CR)], out_hbm.at[pl.ds(lo, _CR)])


def kernel(subject_ids):
    ids = jnp.reshape(subject_ids, (-1,)).astype(jnp.int32)
    return _mask_kernel(ids)
